# baseline (device time: 427999 ns/iter reference)
import jax
import jax.numpy as jnp
from jax import lax
from jax.experimental import pallas as pl
from jax.experimental.pallas import tpu as pltpu

N_DEV = 16
SQ = 256
D_MODEL = 1024
SKV = 4096
H_PER = 8
DH = 128
ROWS = SQ // N_DEV
SCALE = 0.08838834764831843


def _bf(x):
    return x.astype(jnp.bfloat16)


N_CHUNK = 8
CH = SKV // N_CHUNK
NEG_INF = -1e30


def _fused_body(x_ref, wq_ref, wo_ref, k_hbm, v_hbm, out_ref,
                k_slab, v_slab, part_bf, red_bf, p1_buf,
                copy_sems, p1_sems, p2_sems, send_sems):
    me = lax.axis_index("i")
    col0 = me * (H_PER * DH)

    kcps, vcps = [], []
    for c in range(N_CHUNK):
        kcp = pltpu.make_async_copy(
            k_hbm.at[pl.ds(c * CH, CH), pl.ds(col0, H_PER * DH)],
            k_slab.at[pl.ds(c * CH, CH), :], copy_sems.at[0, c])
        vcp = pltpu.make_async_copy(
            v_hbm.at[pl.ds(c * CH, CH), pl.ds(col0, H_PER * DH)],
            v_slab.at[pl.ds(c * CH, CH), :], copy_sems.at[1, c])
        kcp.start()
        vcp.start()
        kcps.append(kcp)
        vcps.append(vcp)

    q = jnp.dot(_bf(x_ref[0]), _bf(wq_ref[...]),
                preferred_element_type=jnp.float32)
    qbf = _bf(q)

    ms = [jnp.full((SQ, 1), NEG_INF, jnp.float32) for _ in range(H_PER)]
    ls = [jnp.zeros((SQ, 1), jnp.float32) for _ in range(H_PER)]
    accs = [jnp.zeros((SQ, DH), jnp.float32) for _ in range(H_PER)]

    for c in range(N_CHUNK):
        kcps[c].wait()
        vcps[c].wait()
        kc = _bf(k_slab[pl.ds(c * CH, CH), :])
        vc = _bf(v_slab[pl.ds(c * CH, CH), :])
        for h in range(H_PER):
            qh = qbf[:, h * DH:(h + 1) * DH]
            kh = kc[:, h * DH:(h + 1) * DH]
            vh = vc[:, h * DH:(h + 1) * DH]
            s = lax.dot_general(
                qh, kh, (((1,), (1,)), ((), ())),
                preferred_element_type=jnp.float32) * SCALE
            mc = jnp.max(s, axis=1, keepdims=True)
            m_new = jnp.maximum(ms[h], mc)
            alpha = jnp.exp(ms[h] - m_new)
            p = jnp.exp(s - m_new)
            ls[h] = ls[h] * alpha + jnp.sum(p, axis=1, keepdims=True)
            accs[h] = accs[h] * alpha + jnp.dot(
                _bf(p), vh, preferred_element_type=jnp.float32)
            ms[h] = m_new

    partial = jnp.zeros((SQ, D_MODEL), jnp.float32)
    for h in range(H_PER):
        oh = accs[h] / ls[h]
        partial = partial + jnp.dot(
            _bf(oh), _bf(wo_ref[pl.ds(h * DH, DH), :]),
            preferred_element_type=jnp.float32)
    part_bf[...] = _bf(partial)

    p1_descs = []
    for k in range(1, N_DEV):
        peer = lax.rem(me + k, N_DEV)
        d = pltpu.make_async_remote_copy(
            src_ref=part_bf.at[pl.ds(peer * ROWS, ROWS), :],
            dst_ref=p1_buf.at[N_DEV - k],
            send_sem=send_sems.at[0, k],
            recv_sem=p1_sems.at[N_DEV - k],
            device_id=(peer,),
            device_id_type=pl.DeviceIdType.MESH,
        )
        d.start()
        p1_descs.append(d)

    acc = part_bf[pl.ds(me * ROWS, ROWS), :].astype(jnp.float32)
    for k in range(1, N_DEV):
        r = pltpu.make_async_remote_copy(
            src_ref=part_bf.at[pl.ds(0, ROWS), :],
            dst_ref=p1_buf.at[k],
            send_sem=send_sems.at[0, 0],
            recv_sem=p1_sems.at[k],
            device_id=(me,),
            device_id_type=pl.DeviceIdType.MESH,
        )
        r.wait_recv()
        acc = acc + p1_buf[k].astype(jnp.float32)
    red_bf[pl.ds(me * ROWS, ROWS), :] = _bf(acc)

    p2_descs = []
    for k in range(1, N_DEV):
        peer = lax.rem(me + k, N_DEV)
        d = pltpu.make_async_remote_copy(
            src_ref=red_bf.at[pl.ds(me * ROWS, ROWS), :],
            dst_ref=red_bf.at[pl.ds(me * ROWS, ROWS), :],
            send_sem=send_sems.at[1, k],
            recv_sem=p2_sems.at[N_DEV - k],
            device_id=(peer,),
            device_id_type=pl.DeviceIdType.MESH,
        )
        d.start()
        p2_descs.append(d)

    for k in range(1, N_DEV):
        src_chunk = lax.rem(me + k, N_DEV)
        r = pltpu.make_async_remote_copy(
            src_ref=red_bf.at[pl.ds(0, ROWS), :],
            dst_ref=red_bf.at[pl.ds(src_chunk * ROWS, ROWS), :],
            send_sem=send_sems.at[1, 0],
            recv_sem=p2_sems.at[k],
            device_id=(me,),
            device_id_type=pl.DeviceIdType.MESH,
        )
        r.wait_recv()

    out_ref[0] = red_bf[...].astype(jnp.float32)

    for d in p1_descs + p2_descs:
        d.wait_send()


def kernel(x, Wq, Wo, K_ext, V_ext):
    K2 = K_ext.reshape(SKV, 16 * H_PER * DH)
    V2 = V_ext.reshape(SKV, 16 * H_PER * DH)
    return pl.pallas_call(
        _fused_body,
        out_shape=jax.ShapeDtypeStruct((1, SQ, D_MODEL), jnp.float32),
        in_specs=[
            pl.BlockSpec(memory_space=pltpu.VMEM),
            pl.BlockSpec(memory_space=pltpu.VMEM),
            pl.BlockSpec(memory_space=pltpu.VMEM),
            pl.BlockSpec(memory_space=pltpu.MemorySpace.HBM),
            pl.BlockSpec(memory_space=pltpu.MemorySpace.HBM),
        ],
        out_specs=pl.BlockSpec(memory_space=pltpu.VMEM),
        scratch_shapes=[
            pltpu.VMEM((SKV, H_PER * DH), jnp.float32),
            pltpu.VMEM((SKV, H_PER * DH), jnp.float32),
            pltpu.VMEM((SQ, D_MODEL), jnp.bfloat16),
            pltpu.VMEM((SQ, D_MODEL), jnp.bfloat16),
            pltpu.VMEM((N_DEV, ROWS, D_MODEL), jnp.bfloat16),
            pltpu.SemaphoreType.DMA((2, N_CHUNK)),
            pltpu.SemaphoreType.DMA((N_DEV,)),
            pltpu.SemaphoreType.DMA((N_DEV,)),
            pltpu.SemaphoreType.DMA((2, N_DEV)),
        ],
        compiler_params=pltpu.CompilerParams(
            vmem_limit_bytes=100 * 1024 * 1024),
    )(x, Wq, Wo, K2, V2)


# device time: 52696 ns/iter; 8.1220x vs baseline; 8.1220x over previous
import jax
import jax.numpy as jnp
from jax import lax
from jax.experimental import pallas as pl
from jax.experimental.pallas import tpu as pltpu

N_DEV = 16
SQ = 256
D_MODEL = 1024
SKV = 4096
H_PER = 8
DH = 128
ROWS = SQ // N_DEV
SCALE = 0.08838834764831843


def _bf(x):
    return x.astype(jnp.bfloat16)


def _fused_body(x_ref, wq_ref, wo_ref, k_hbm, v_hbm, out_ref,
                k_bufs, v_bufs, part_bf, red_bf, p1_buf,
                copy_sems, p1_sems, p2_sems, send_sems):
    me = lax.axis_index("i")
    head0 = me * H_PER

    HALF = SKV // 2
    kcps, vcps = [], []
    for h in range(H_PER):
        for half in range(2):
            kcp = pltpu.make_async_copy(
                k_hbm.at[0, pl.ds(half * HALF, HALF), head0 + h, :],
                k_bufs.at[h, pl.ds(half * HALF, HALF), :],
                copy_sems.at[0, 2 * h + half])
            vcp = pltpu.make_async_copy(
                v_hbm.at[0, pl.ds(half * HALF, HALF), head0 + h, :],
                v_bufs.at[h, pl.ds(half * HALF, HALF), :],
                copy_sems.at[1, 2 * h + half])
            kcp.start()
            vcp.start()
            kcps.append(kcp)
            vcps.append(vcp)

    q = jnp.dot(_bf(x_ref[0]), _bf(wq_ref[...]),
                preferred_element_type=jnp.float32)

    partial = jnp.zeros((SQ, D_MODEL), jnp.float32)
    for h in range(H_PER):
        for half in range(2):
            kcps[2 * h + half].wait()
            vcps[2 * h + half].wait()
        qh = _bf(q[:, h * DH:(h + 1) * DH])
        kh = _bf(k_bufs[h])
        vh = _bf(v_bufs[h])
        s = lax.dot_general(
            qh, kh, (((1,), (1,)), ((), ())),
            preferred_element_type=jnp.float32) * SCALE
        m = jnp.max(s, axis=1, keepdims=True)
        p = jnp.exp(s - m)
        l = jnp.sum(p, axis=1, keepdims=True)
        oh = jnp.dot(_bf(p), vh, preferred_element_type=jnp.float32) / l
        partial = partial + jnp.dot(
            _bf(oh), _bf(wo_ref[pl.ds(h * DH, DH), :]),
            preferred_element_type=jnp.float32)
    part_bf[...] = _bf(partial)

    p1_descs = []
    for k in range(1, N_DEV):
        peer = lax.rem(me + k, N_DEV)
        d = pltpu.make_async_remote_copy(
            src_ref=part_bf.at[pl.ds(peer * ROWS, ROWS), :],
            dst_ref=p1_buf.at[N_DEV - k],
            send_sem=send_sems.at[0, k],
            recv_sem=p1_sems.at[N_DEV - k],
            device_id=(peer,),
            device_id_type=pl.DeviceIdType.MESH,
        )
        d.start()
        p1_descs.append(d)

    acc = part_bf[pl.ds(me * ROWS, ROWS), :].astype(jnp.float32)
    for k in range(1, N_DEV):
        r = pltpu.make_async_remote_copy(
            src_ref=part_bf.at[pl.ds(0, ROWS), :],
            dst_ref=p1_buf.at[k],
            send_sem=send_sems.at[0, 0],
            recv_sem=p1_sems.at[k],
            device_id=(me,),
            device_id_type=pl.DeviceIdType.MESH,
        )
        r.wait_recv()
        acc = acc + p1_buf[k].astype(jnp.float32)
    red_bf[pl.ds(me * ROWS, ROWS), :] = _bf(acc)

    p2_descs = []
    for k in range(1, N_DEV):
        peer = lax.rem(me + k, N_DEV)
        d = pltpu.make_async_remote_copy(
            src_ref=red_bf.at[pl.ds(me * ROWS, ROWS), :],
            dst_ref=red_bf.at[pl.ds(me * ROWS, ROWS), :],
            send_sem=send_sems.at[1, k],
            recv_sem=p2_sems.at[N_DEV - k],
            device_id=(peer,),
            device_id_type=pl.DeviceIdType.MESH,
        )
        d.start()
        p2_descs.append(d)

    for k in range(1, N_DEV):
        src_chunk = lax.rem(me + k, N_DEV)
        r = pltpu.make_async_remote_copy(
            src_ref=red_bf.at[pl.ds(0, ROWS), :],
            dst_ref=red_bf.at[pl.ds(src_chunk * ROWS, ROWS), :],
            send_sem=send_sems.at[1, 0],
            recv_sem=p2_sems.at[k],
            device_id=(me,),
            device_id_type=pl.DeviceIdType.MESH,
        )
        r.wait_recv()

    out_ref[0] = red_bf[...].astype(jnp.float32)

    for d in p1_descs + p2_descs:
        d.wait_send()


def kernel(x, Wq, Wo, K_ext, V_ext):
    return pl.pallas_call(
        _fused_body,
        out_shape=jax.ShapeDtypeStruct((1, SQ, D_MODEL), jnp.float32),
        in_specs=[
            pl.BlockSpec(memory_space=pltpu.VMEM),
            pl.BlockSpec(memory_space=pltpu.VMEM),
            pl.BlockSpec(memory_space=pltpu.VMEM),
            pl.BlockSpec(memory_space=pltpu.MemorySpace.HBM),
            pl.BlockSpec(memory_space=pltpu.MemorySpace.HBM),
        ],
        out_specs=pl.BlockSpec(memory_space=pltpu.VMEM),
        scratch_shapes=[
            pltpu.VMEM((H_PER, SKV, DH), jnp.float32),
            pltpu.VMEM((H_PER, SKV, DH), jnp.float32),
            pltpu.VMEM((SQ, D_MODEL), jnp.bfloat16),
            pltpu.VMEM((SQ, D_MODEL), jnp.bfloat16),
            pltpu.VMEM((N_DEV, ROWS, D_MODEL), jnp.bfloat16),
            pltpu.SemaphoreType.DMA((2, 2 * H_PER)),
            pltpu.SemaphoreType.DMA((N_DEV,)),
            pltpu.SemaphoreType.DMA((N_DEV,)),
            pltpu.SemaphoreType.DMA((2, N_DEV)),
        ],
        compiler_params=pltpu.CompilerParams(
            vmem_limit_bytes=100 * 1024 * 1024),
    )(x, Wq, Wo, K_ext, V_ext)
